# baseline (device time: 38129 ns/iter reference)
import functools

import jax
import jax.numpy as jnp
from jax import lax
from jax.experimental import pallas as pl
from jax.experimental.pallas import tpu as pltpu

N_DEV = 4
B = 8
H = 8
D = 128
BS = 16
NB = 512
PP = 512
R = B * H
PC = 64
CC = PC * BS * H
NC = PP // PC
NEG = -1e30


def _body(q_ref, k_ref, v_ref, bt_ref, lens_ref, out_ref,
          q_scr, lcx_scr, hpen_scr, l_scr, o_scr,
          o_comm, l_comm, send_sems, recv_sems):
    my = lax.axis_index("i")
    c_id = pl.program_id(0)

    @pl.when(c_id == 0)
    def _init():
        off = my * PP
        q_scr[...] = (q_ref[...] * (D ** -0.5)).astype(jnp.bfloat16)

        bt = bt_ref[...]
        lens = lens_ref[...]
        JC = 128
        c = jnp.zeros((B, PP), jnp.float32)
        for j0 in range(0, NB, JC):
            btc = bt[:, j0:j0 + JC]
            jio = lax.broadcasted_iota(jnp.int32, (B, JC, PP), 1) + j0
            pio = lax.broadcasted_iota(jnp.int32, (B, JC, PP), 2)
            hitc = jnp.where(
                (btc[:, :, None] == pio + off) & (jio < lens[:, :, None]),
                1.0, 0.0,
            )
            c = c + jnp.sum(hitc, axis=1)

        hio = lax.broadcasted_iota(jnp.int32, (H, CC), 0)
        cio = lax.rem(lax.broadcasted_iota(jnp.int32, (H, CC), 1), H)
        hpen_scr[...] = jnp.where(hio == cio, -12.0, -1e9)

        pro = lax.broadcasted_iota(jnp.int32, (PC, CC), 0)
        cco = lax.broadcasted_iota(jnp.int32, (PC, CC), 1)
        exp2 = jnp.where(
            pro == cco // (BS * H), 1.0, 0.0
        ).astype(jnp.bfloat16)
        for cc in range(NC):
            ckx = lax.dot_general(
                c[:, cc * PC:(cc + 1) * PC].astype(jnp.bfloat16), exp2,
                (((1,), (0,)), ((), ())),
                preferred_element_type=jnp.float32,
            )
            lcx_scr[:, cc * CC:(cc + 1) * CC] = jnp.log(
                jnp.maximum(ckx, 1e-30)
            )

        l_scr[...] = jnp.zeros((B, H, 1), jnp.float32)
        o_scr[...] = jnp.zeros((R, D), jnp.float32)

    k_c = k_ref[...].astype(jnp.bfloat16)
    s = lax.dot_general(
        q_scr[...], k_c, (((1,), (1,)), ((), ())),
        preferred_element_type=jnp.float32,
    )
    lcx = lcx_scr[:, pl.ds(c_id * CC, CC)]
    p_c = jnp.exp(s.reshape(B, H, CC) + lcx[:, None, :]
                  + hpen_scr[...][None, :, :]
                  ).astype(jnp.bfloat16)
    pv = lax.dot_general(
        p_c.reshape(R, CC), v_ref[...].astype(jnp.bfloat16),
        (((1,), (0,)), ((), ())),
        preferred_element_type=jnp.float32,
    )
    l_scr[...] = l_scr[...] + jnp.sum(
        p_c.astype(jnp.float32), axis=2, keepdims=True)
    o_scr[...] = o_scr[...] + pv

    @pl.when(c_id == NC - 1)
    def _finish():
        o_comm[my] = o_scr[...]
        l_comm[my] = l_scr[...].reshape(R, 1)

        bar = pltpu.get_barrier_semaphore()
        for dlt in range(1, N_DEV):
            tgt = lax.rem(my + dlt, N_DEV)
            pl.semaphore_signal(bar, inc=1, device_id=(tgt,),
                                device_id_type=pl.DeviceIdType.MESH)
        pl.semaphore_wait(bar, N_DEV - 1)

        sends = []
        for dlt in range(1, N_DEV):
            tgt = lax.rem(my + dlt, N_DEV)
            for t, buf in ((0, o_comm), (1, l_comm)):
                r = pltpu.make_async_remote_copy(
                    src_ref=buf.at[my], dst_ref=buf.at[my],
                    send_sem=send_sems.at[dlt - 1, t],
                    recv_sem=recv_sems.at[my, t],
                    device_id=(tgt,), device_id_type=pl.DeviceIdType.MESH,
                )
                r.start()
                sends.append(r)

        for dlt in range(1, N_DEV):
            src = lax.rem(my + dlt, N_DEV)
            for t, buf in ((0, o_comm), (1, l_comm)):
                rw = pltpu.make_async_remote_copy(
                    src_ref=buf.at[src], dst_ref=buf.at[src],
                    send_sem=send_sems.at[dlt - 1, t],
                    recv_sem=recv_sems.at[src, t],
                    device_id=(src,), device_id_type=pl.DeviceIdType.MESH,
                )
                rw.wait_recv()
        for r in sends:
            r.wait_send()

        lg = jnp.sum(l_comm[...], axis=0)
        onum = jnp.sum(o_comm[...], axis=0)
        og = onum / lg
        out_ref[:, 0] = og.reshape(B, H, D)

        @functools.partial(pl.run_scoped,
                           exit_sem=pltpu.SemaphoreType.REGULAR)
        def _(exit_sem):
            for dlt in range(1, N_DEV):
                tgt = lax.rem(my + dlt, N_DEV)
                pl.semaphore_signal(exit_sem, inc=1, device_id=(tgt,),
                                    device_id_type=pl.DeviceIdType.MESH)
            pl.semaphore_wait(exit_sem, N_DEV - 1)


def kernel(Q, K, V, bt, lens):
    lens2 = lens.reshape(B, 1)
    q2 = Q.reshape(R, D)
    k2 = K.reshape(PP * BS * H, D)
    v2 = V.reshape(PP * BS * H, D)

    return pl.pallas_call(
        _body,
        grid=(NC,),
        out_shape=jax.ShapeDtypeStruct((B, 1, H, D), jnp.float32),
        in_specs=[
            pl.BlockSpec((R, D), lambda c: (0, 0)),
            pl.BlockSpec((CC, D), lambda c: (c, 0)),
            pl.BlockSpec((CC, D), lambda c: (c, 0)),
            pl.BlockSpec((B, NB), lambda c: (0, 0)),
            pl.BlockSpec((B, 1), lambda c: (0, 0)),
        ],
        out_specs=pl.BlockSpec((B, 1, H, D), lambda c: (0, 0, 0, 0)),
        scratch_shapes=[
            pltpu.VMEM((R, D), jnp.bfloat16),
            pltpu.VMEM((B, NC * CC), jnp.float32),
            pltpu.VMEM((H, CC), jnp.float32),
            pltpu.VMEM((B, H, 1), jnp.float32),
            pltpu.VMEM((R, D), jnp.float32),
            pltpu.VMEM((N_DEV, R, D), jnp.float32),
            pltpu.VMEM((N_DEV, R, 1), jnp.float32),
            pltpu.SemaphoreType.DMA((N_DEV - 1, 2)),
            pltpu.SemaphoreType.DMA((N_DEV, 2)),
        ],
        compiler_params=pltpu.CompilerParams(
            collective_id=0,
            vmem_limit_bytes=60 * 1024 * 1024,
        ),
    )(q2, k2, v2, bt, lens2)


# device time: 37610 ns/iter; 1.0138x vs baseline; 1.0138x over previous
import functools

import jax
import jax.numpy as jnp
from jax import lax
from jax.experimental import pallas as pl
from jax.experimental.pallas import tpu as pltpu

N_DEV = 4
B = 8
H = 8
D = 128
BS = 16
NB = 512
PP = 512
R = B * H
PC = 64
CC = PC * BS * H
NC = PP // PC


def _body(q_ref, k_hbm, v_hbm, bt_ref, lens_ref, out_ref,
          k_buf, v_buf, copy_sems,
          lcx_scr, hpen_scr, l_scr, o_scr,
          o_comm, l_comm, send_sems, recv_sems):
    my = lax.axis_index("i")

    def start_copy(c, slot):
        kd = pltpu.make_async_copy(
            k_hbm.at[pl.ds(c * CC, CC), :], k_buf.at[slot],
            copy_sems.at[slot, 0])
        vd = pltpu.make_async_copy(
            v_hbm.at[pl.ds(c * CC, CC), :], v_buf.at[slot],
            copy_sems.at[slot, 1])
        kd.start()
        vd.start()
        return kd, vd

    pend = [start_copy(0, 0), start_copy(1, 1)]

    q = (q_ref[...] * (D ** -0.5)).astype(jnp.bfloat16)

    off = my * PP
    bt = bt_ref[...]
    lens = lens_ref[...]
    JC = 128
    c = jnp.zeros((B, PP), jnp.float32)
    for j0 in range(0, NB, JC):
        btc = bt[:, j0:j0 + JC]
        jio = lax.broadcasted_iota(jnp.int32, (B, JC, PP), 1) + j0
        pio = lax.broadcasted_iota(jnp.int32, (B, JC, PP), 2)
        hitc = jnp.where(
            (btc[:, :, None] == pio + off) & (jio < lens[:, :, None]),
            1.0, 0.0,
        )
        c = c + jnp.sum(hitc, axis=1)

    hio = lax.broadcasted_iota(jnp.int32, (H, CC), 0)
    cio = lax.rem(lax.broadcasted_iota(jnp.int32, (H, CC), 1), H)
    hpen_scr[...] = jnp.where(hio == cio, -12.0, -1e9)

    pro = lax.broadcasted_iota(jnp.int32, (PC, CC), 0)
    cco = lax.broadcasted_iota(jnp.int32, (PC, CC), 1)
    exp2 = jnp.where(pro == cco // (BS * H), 1.0, 0.0).astype(jnp.bfloat16)
    for cc in range(NC):
        ckx = lax.dot_general(
            c[:, cc * PC:(cc + 1) * PC].astype(jnp.bfloat16), exp2,
            (((1,), (0,)), ((), ())),
            preferred_element_type=jnp.float32,
        )
        lcx_scr[:, cc * CC:(cc + 1) * CC] = jnp.log(
            jnp.maximum(ckx, 1e-30))

    l_scr[...] = jnp.zeros((B, H, 1), jnp.float32)
    o_scr[...] = jnp.zeros((R, D), jnp.float32)

    for cc in range(NC):
        slot = cc % 2
        kd, vd = pend[slot]
        kd.wait()
        vd.wait()
        k_c = k_buf[slot].astype(jnp.bfloat16)
        s = lax.dot_general(
            q, k_c, (((1,), (1,)), ((), ())),
            preferred_element_type=jnp.float32,
        )
        lcx = lcx_scr[:, cc * CC:(cc + 1) * CC]
        p_c = jnp.exp(s.reshape(B, H, CC) + lcx[:, None, :]
                      + hpen_scr[...][None, :, :]
                      ).astype(jnp.bfloat16)
        pv = lax.dot_general(
            p_c.reshape(R, CC), v_buf[slot].astype(jnp.bfloat16),
            (((1,), (0,)), ((), ())),
            preferred_element_type=jnp.float32,
        )
        l_scr[...] = l_scr[...] + jnp.sum(
            p_c.astype(jnp.float32), axis=2, keepdims=True)
        o_scr[...] = o_scr[...] + pv
        if cc + 2 < NC:
            pend[slot] = start_copy(cc + 2, slot)

    o_comm[my] = o_scr[...]
    l_comm[my] = l_scr[...].reshape(R, 1)

    bar = pltpu.get_barrier_semaphore()
    for dlt in range(1, N_DEV):
        tgt = lax.rem(my + dlt, N_DEV)
        pl.semaphore_signal(bar, inc=1, device_id=(tgt,),
                            device_id_type=pl.DeviceIdType.MESH)
    pl.semaphore_wait(bar, N_DEV - 1)

    sends = []
    for dlt in range(1, N_DEV):
        tgt = lax.rem(my + dlt, N_DEV)
        for t, buf in ((0, o_comm), (1, l_comm)):
            r = pltpu.make_async_remote_copy(
                src_ref=buf.at[my], dst_ref=buf.at[my],
                send_sem=send_sems.at[dlt - 1, t],
                recv_sem=recv_sems.at[my, t],
                device_id=(tgt,), device_id_type=pl.DeviceIdType.MESH,
            )
            r.start()
            sends.append(r)

    for dlt in range(1, N_DEV):
        src = lax.rem(my + dlt, N_DEV)
        for t, buf in ((0, o_comm), (1, l_comm)):
            rw = pltpu.make_async_remote_copy(
                src_ref=buf.at[src], dst_ref=buf.at[src],
                send_sem=send_sems.at[dlt - 1, t],
                recv_sem=recv_sems.at[src, t],
                device_id=(src,), device_id_type=pl.DeviceIdType.MESH,
            )
            rw.wait_recv()
    for r in sends:
        r.wait_send()

    lg = jnp.sum(l_comm[...], axis=0)
    onum = jnp.sum(o_comm[...], axis=0)
    og = onum / lg
    out_ref[:, 0] = og.reshape(B, H, D)

    @functools.partial(pl.run_scoped,
                       exit_sem=pltpu.SemaphoreType.REGULAR)
    def _(exit_sem):
        for dlt in range(1, N_DEV):
            tgt = lax.rem(my + dlt, N_DEV)
            pl.semaphore_signal(exit_sem, inc=1, device_id=(tgt,),
                                device_id_type=pl.DeviceIdType.MESH)
        pl.semaphore_wait(exit_sem, N_DEV - 1)


def kernel(Q, K, V, bt, lens):
    lens2 = lens.reshape(B, 1)
    q2 = Q.reshape(R, D)
    k2 = K.reshape(PP * BS * H, D)
    v2 = V.reshape(PP * BS * H, D)

    return pl.pallas_call(
        _body,
        out_shape=jax.ShapeDtypeStruct((B, 1, H, D), jnp.float32),
        in_specs=[
            pl.BlockSpec(memory_space=pltpu.VMEM),
            pl.BlockSpec(memory_space=pl.ANY),
            pl.BlockSpec(memory_space=pl.ANY),
            pl.BlockSpec(memory_space=pltpu.VMEM),
            pl.BlockSpec(memory_space=pltpu.VMEM),
        ],
        out_specs=pl.BlockSpec(memory_space=pltpu.VMEM),
        scratch_shapes=[
            pltpu.VMEM((2, CC, D), jnp.float32),
            pltpu.VMEM((2, CC, D), jnp.float32),
            pltpu.SemaphoreType.DMA((2, 2)),
            pltpu.VMEM((B, NC * CC), jnp.float32),
            pltpu.VMEM((H, CC), jnp.float32),
            pltpu.VMEM((B, H, 1), jnp.float32),
            pltpu.VMEM((R, D), jnp.float32),
            pltpu.VMEM((N_DEV, R, D), jnp.float32),
            pltpu.VMEM((N_DEV, R, 1), jnp.float32),
            pltpu.SemaphoreType.DMA((N_DEV - 1, 2)),
            pltpu.SemaphoreType.DMA((N_DEV, 2)),
        ],
        compiler_params=pltpu.CompilerParams(
            collective_id=0,
            vmem_limit_bytes=60 * 1024 * 1024,
        ),
    )(q2, k2, v2, bt, lens2)
